# baseline (speedup 1.0000x reference)
import functools

import jax
import jax.numpy as jnp
from jax import lax
from jax.experimental import pallas as pl
from jax.experimental.pallas import tpu as pltpu
from jax.experimental.pallas import tpu_sc as plsc

N = 10000
E = 320000
NPAD = 10016
NC, NS = 2, 16
NW = NC * NS
CHUNK = 128
NBUF = 4
LAG = 2
K1 = 80
EPT = K1 * CHUNK
EPAD = EPT * NW

_mesh = plsc.VectorSubcoreMesh(
    core_axis_name="c", subcore_axis_name="s", num_cores=NC, num_subcores=NS)
_sc_params = pltpu.CompilerParams(use_tc_tiling_on_sc=False)


def _deg_body(dst_hbm, zeros_hbm, ones_hbm, out_hbm, idx_v, ones_v, acc_sh, sem):
    cid = lax.axis_index("c")
    sid = lax.axis_index("s")
    wid = cid * NS + sid
    pltpu.sync_copy(dst_hbm.at[wid], idx_v)
    pltpu.sync_copy(ones_hbm, ones_v)

    @pl.when(sid == 0)
    def _zero():
        pltpu.sync_copy(zeros_hbm, acc_sh)

    plsc.subcore_barrier()

    def body(j, carry):
        pltpu.async_copy(ones_v, acc_sh.at[idx_v.at[j]], sem, add=True)
        return carry

    lax.fori_loop(0, K1, body, 0)

    def drain(j, carry):
        pltpu.make_async_copy(ones_v, acc_sh.at[idx_v.at[0]], sem).wait()
        return carry

    lax.fori_loop(0, K1, drain, 0)
    plsc.subcore_barrier()

    @pl.when(sid == 0)
    def _flush():
        pltpu.sync_copy(acc_sh, out_hbm.at[cid])


_deg_kernel = functools.partial(
    pl.kernel,
    out_type=jax.ShapeDtypeStruct((NC, NPAD, 16), jnp.float32),
    mesh=_mesh,
    compiler_params=_sc_params,
    scratch_types=[
        pltpu.VMEM((K1, CHUNK), jnp.int32),
        pltpu.VMEM((CHUNK, 16), jnp.float32),
        pltpu.VMEM_SHARED((NPAD, 16), jnp.float32),
        pltpu.SemaphoreType.DMA,
    ],
)(_deg_body)


def _make_scatter(D):
    def body(src_hbm, dst_hbm, y_hbm, zeros_hbm, out_hbm,
             src_v, dst_v, rows_v, acc_sh, *sems):
        gsems, ssems = sems[:NBUF], sems[NBUF:]
        cid = lax.axis_index("c")
        sid = lax.axis_index("s")
        wid = cid * NS + sid
        pltpu.sync_copy(src_hbm.at[wid], src_v)
        pltpu.sync_copy(dst_hbm.at[wid], dst_v)
        for g in range(LAG):
            pltpu.async_copy(y_hbm.at[src_v.at[g]], rows_v.at[g], gsems[g])

        @pl.when(sid == 0)
        def _zero():
            pltpu.sync_copy(zeros_hbm, acc_sh)

        plsc.subcore_barrier()

        def step(g, carry):
            for b in range(NBUF):
                j = g * NBUF + b
                pltpu.make_async_copy(
                    y_hbm.at[src_v.at[j]], rows_v.at[b], gsems[b]).wait()
                pltpu.async_copy(
                    rows_v.at[b], acc_sh.at[dst_v.at[j]], ssems[b], add=True)
                b2 = (b + LAG) % NBUF

                @pl.when(j + LAG < K1)
                def _refill():
                    @pl.when(j - LAG >= 0)
                    def _wait_prev_scatter():
                        pltpu.make_async_copy(
                            rows_v.at[b2], acc_sh.at[dst_v.at[0]],
                            ssems[b2]).wait()

                    pltpu.async_copy(
                        y_hbm.at[src_v.at[j + LAG]], rows_v.at[b2], gsems[b2])
            return carry

        lax.fori_loop(0, K1 // NBUF, step, 0)
        for b in range(NBUF):
            pltpu.make_async_copy(
                rows_v.at[b], acc_sh.at[dst_v.at[0]], ssems[b]).wait()
        plsc.subcore_barrier()

        @pl.when(sid == 0)
        def _flush():
            pltpu.sync_copy(acc_sh, out_hbm.at[cid])

    return functools.partial(
        pl.kernel,
        out_type=jax.ShapeDtypeStruct((NC, NPAD, D), jnp.float32),
        mesh=_mesh,
        compiler_params=_sc_params,
        scratch_types=[
            pltpu.VMEM((K1, CHUNK), jnp.int32),
            pltpu.VMEM((K1, CHUNK), jnp.int32),
            pltpu.VMEM((NBUF, CHUNK, D), jnp.float32),
            pltpu.VMEM_SHARED((NPAD, D), jnp.float32),
        ] + [pltpu.SemaphoreType.DMA] * (2 * NBUF),
    )(body)


_scatter64 = _make_scatter(64)
_scatter48 = _make_scatter(48)


def _dis(da_ref, db_ref):
    deg = da_ref[:, 0:1] + db_ref[:, 0:1] + 1.0
    return lax.rsqrt(deg)


def _tc1_body(x_ref, w_ref, da_ref, db_ref, y_ref):
    dis = _dis(da_ref, db_ref)
    xw = jnp.dot(x_ref[...], w_ref[...], preferred_element_type=jnp.float32)
    y_ref[...] = xw * dis


def _tc2_body(za_ref, zb_ref, y1_ref, da_ref, db_ref, w_ref, b1_ref, y2_ref):
    dis = _dis(da_ref, db_ref)
    pre = (za_ref[...] + zb_ref[...] + y1_ref[...]) * dis + b1_ref[...]
    h = jnp.maximum(pre, 0.0)
    rows = lax.broadcasted_iota(jnp.int32, (NPAD, 1), 0)
    h = jnp.where(rows < N, h, 0.0)
    y2_ref[...] = jnp.dot(h, w_ref[...], preferred_element_type=jnp.float32) * dis


def _tc3_body(za_ref, zb_ref, y2_ref, da_ref, db_ref, b2_ref, out_ref):
    dis = _dis(da_ref, db_ref)
    o = (za_ref[...] + zb_ref[...] + y2_ref[...]) * dis + b2_ref[...]
    cols = lax.broadcasted_iota(jnp.int32, (NPAD, 48), 1)
    valid = cols < 40
    m = jnp.max(jnp.where(valid, o, -jnp.inf), axis=1, keepdims=True)
    e = jnp.where(valid, jnp.exp(o - m), 0.0)
    s = jnp.sum(e, axis=1, keepdims=True)
    ls = o - m - jnp.log(s)
    out_ref[...] = ls[:N, :40]


def _tc_call(body, out_shape, *args):
    return pl.pallas_call(body, out_shape=out_shape)(*args)


def kernel(x, edge_index, W1, b1, W2, b2):
    src = edge_index[0].astype(jnp.int32)
    dst = edge_index[1].astype(jnp.int32)
    pad = jnp.full((EPAD - E,), N, dtype=jnp.int32)
    src_r = jnp.concatenate([src, pad]).reshape(NW, K1, CHUNK)
    dst_r = jnp.concatenate([dst, pad]).reshape(NW, K1, CHUNK)

    x_p = jnp.pad(x, ((0, NPAD - N), (0, 0)))
    w2_p = jnp.pad(W2, ((0, 0), (0, 8)))
    b1_r = b1.reshape(1, 64)
    b2_r = jnp.pad(b2, (0, 8)).reshape(1, 48)
    zeros16 = jnp.zeros((NPAD, 16), jnp.float32)
    zeros64 = jnp.zeros((NPAD, 64), jnp.float32)
    zeros48 = jnp.zeros((NPAD, 48), jnp.float32)
    ones16 = jnp.ones((CHUNK, 16), jnp.float32)

    dp = _deg_kernel(dst_r, zeros16, ones16)
    da, db = dp[0], dp[1]

    y1 = _tc_call(_tc1_body, jax.ShapeDtypeStruct((NPAD, 64), jnp.float32),
                  x_p, W1, da, db)
    z1 = _scatter64(src_r, dst_r, y1, zeros64)
    y2 = _tc_call(_tc2_body, jax.ShapeDtypeStruct((NPAD, 48), jnp.float32),
                  z1[0], z1[1], y1, da, db, w2_p, b1_r)
    z2 = _scatter48(src_r, dst_r, y2, zeros48)
    out = _tc_call(_tc3_body, jax.ShapeDtypeStruct((N, 40), jnp.float32),
                   z2[0], z2[1], y2, da, db, b2_r)
    return out


# speedup vs baseline: 1.8721x; 1.8721x over previous
import functools

import jax
import jax.numpy as jnp
from jax import lax
from jax.experimental import pallas as pl
from jax.experimental.pallas import tpu as pltpu
from jax.experimental.pallas import tpu_sc as plsc

N = 10000
E = 320000
NPAD = 10016
NC, NS = 2, 16
NW = NC * NS
CHUNK = 128
NBUF = 2
LAG = 1
K1 = 80
EPT = K1 * CHUNK
EPAD = EPT * NW

_mesh = plsc.VectorSubcoreMesh(
    core_axis_name="c", subcore_axis_name="s", num_cores=NC, num_subcores=NS)
_sc_params = pltpu.CompilerParams(use_tc_tiling_on_sc=False)


def _deg_body(dst_hbm, zeros_hbm, ones_hbm, out_hbm, idx_v, ones_v, acc_sh, sem):
    cid = lax.axis_index("c")
    sid = lax.axis_index("s")
    wid = cid * NS + sid
    pltpu.sync_copy(dst_hbm.at[wid], idx_v)
    pltpu.sync_copy(ones_hbm, ones_v)

    @pl.when(sid == 0)
    def _zero():
        pltpu.sync_copy(zeros_hbm, acc_sh)

    plsc.subcore_barrier()

    def body(j, carry):
        pltpu.async_copy(ones_v, acc_sh.at[idx_v.at[j]], sem, add=True)
        return carry

    lax.fori_loop(0, K1, body, 0)

    def drain(j, carry):
        pltpu.make_async_copy(ones_v, acc_sh.at[idx_v.at[0]], sem).wait()
        return carry

    lax.fori_loop(0, K1, drain, 0)
    plsc.subcore_barrier()

    @pl.when(sid == 0)
    def _flush():
        pltpu.sync_copy(acc_sh, out_hbm.at[cid])


_deg_kernel = functools.partial(
    pl.kernel,
    out_type=jax.ShapeDtypeStruct((NC, NPAD, 16), jnp.float32),
    mesh=_mesh,
    compiler_params=_sc_params,
    scratch_types=[
        pltpu.VMEM((K1, CHUNK), jnp.int32),
        pltpu.VMEM((CHUNK, 16), jnp.float32),
        pltpu.VMEM_SHARED((NPAD, 16), jnp.float32),
        pltpu.SemaphoreType.DMA,
    ],
)(_deg_body)


def _make_scatter(D):
    def body(src_hbm, dst_hbm, y_hbm, zeros_hbm, out_hbm,
             src_v, dst_v, rows_v, acc_sh, y_sh, *sems):
        gsems, ssems = sems[:NBUF], sems[NBUF:]
        cid = lax.axis_index("c")
        sid = lax.axis_index("s")
        wid = cid * NS + sid
        pltpu.sync_copy(src_hbm.at[wid], src_v)
        pltpu.sync_copy(dst_hbm.at[wid], dst_v)

        @pl.when(sid == 0)
        def _zero():
            pltpu.sync_copy(zeros_hbm, acc_sh)

        @pl.when(sid == 1)
        def _stage():
            pltpu.sync_copy(y_hbm, y_sh)

        plsc.subcore_barrier()
        for g in range(LAG):
            pltpu.async_copy(y_sh.at[src_v.at[g]], rows_v.at[g], gsems[g])

        def step(g, carry):
            for b in range(NBUF):
                j = g * NBUF + b
                pltpu.make_async_copy(
                    y_sh.at[src_v.at[j]], rows_v.at[b], gsems[b]).wait()
                pltpu.async_copy(
                    rows_v.at[b], acc_sh.at[dst_v.at[j]], ssems[b], add=True)
                b2 = (b + LAG) % NBUF

                @pl.when(j + LAG < K1)
                def _refill():
                    @pl.when(j - LAG >= 0)
                    def _wait_prev_scatter():
                        pltpu.make_async_copy(
                            rows_v.at[b2], acc_sh.at[dst_v.at[0]],
                            ssems[b2]).wait()

                    pltpu.async_copy(
                        y_sh.at[src_v.at[j + LAG]], rows_v.at[b2], gsems[b2])
            return carry

        lax.fori_loop(0, K1 // NBUF, step, 0)
        for b in range(NBUF):
            pltpu.make_async_copy(
                rows_v.at[b], acc_sh.at[dst_v.at[0]], ssems[b]).wait()
        plsc.subcore_barrier()

        @pl.when(sid == 0)
        def _flush():
            pltpu.sync_copy(acc_sh, out_hbm.at[cid])

    return functools.partial(
        pl.kernel,
        out_type=jax.ShapeDtypeStruct((NC, NPAD, D), jnp.float32),
        mesh=_mesh,
        compiler_params=_sc_params,
        scratch_types=[
            pltpu.VMEM((K1, CHUNK), jnp.int32),
            pltpu.VMEM((K1, CHUNK), jnp.int32),
            pltpu.VMEM((NBUF, CHUNK, D), jnp.float32),
            pltpu.VMEM_SHARED((NPAD, D), jnp.float32),
            pltpu.VMEM_SHARED((NPAD, D), jnp.float32),
        ] + [pltpu.SemaphoreType.DMA] * (2 * NBUF),
    )(body)


_scatter64 = _make_scatter(64)
_scatter48 = _make_scatter(48)


def _dis(da_ref, db_ref):
    deg = da_ref[:, 0:1] + db_ref[:, 0:1] + 1.0
    return lax.rsqrt(deg)


def _tc1_body(x_ref, w_ref, da_ref, db_ref, y_ref):
    dis = _dis(da_ref, db_ref)
    xw = jnp.dot(x_ref[...], w_ref[...], preferred_element_type=jnp.float32)
    y_ref[...] = xw * dis


def _tc2_body(za_ref, zb_ref, y1_ref, da_ref, db_ref, w_ref, b1_ref, y2_ref):
    dis = _dis(da_ref, db_ref)
    pre = (za_ref[...] + zb_ref[...] + y1_ref[...]) * dis + b1_ref[...]
    h = jnp.maximum(pre, 0.0)
    rows = lax.broadcasted_iota(jnp.int32, (NPAD, 1), 0)
    h = jnp.where(rows < N, h, 0.0)
    y2_ref[...] = jnp.dot(h, w_ref[...], preferred_element_type=jnp.float32) * dis


def _tc3_body(za_ref, zb_ref, y2_ref, da_ref, db_ref, b2_ref, out_ref):
    dis = _dis(da_ref, db_ref)
    o = (za_ref[...] + zb_ref[...] + y2_ref[...]) * dis + b2_ref[...]
    cols = lax.broadcasted_iota(jnp.int32, (NPAD, 48), 1)
    valid = cols < 40
    m = jnp.max(jnp.where(valid, o, -jnp.inf), axis=1, keepdims=True)
    e = jnp.where(valid, jnp.exp(o - m), 0.0)
    s = jnp.sum(e, axis=1, keepdims=True)
    ls = o - m - jnp.log(s)
    out_ref[...] = ls[:N, :40]


def _tc_call(body, out_shape, *args):
    return pl.pallas_call(body, out_shape=out_shape)(*args)


def kernel(x, edge_index, W1, b1, W2, b2):
    src = edge_index[0].astype(jnp.int32)
    dst = edge_index[1].astype(jnp.int32)
    pad = jnp.full((EPAD - E,), N, dtype=jnp.int32)
    src_r = jnp.concatenate([src, pad]).reshape(NW, K1, CHUNK)
    dst_r = jnp.concatenate([dst, pad]).reshape(NW, K1, CHUNK)

    x_p = jnp.pad(x, ((0, NPAD - N), (0, 0)))
    w2_p = jnp.pad(W2, ((0, 0), (0, 8)))
    b1_r = b1.reshape(1, 64)
    b2_r = jnp.pad(b2, (0, 8)).reshape(1, 48)
    zeros16 = jnp.zeros((NPAD, 16), jnp.float32)
    zeros64 = jnp.zeros((NPAD, 64), jnp.float32)
    zeros48 = jnp.zeros((NPAD, 48), jnp.float32)
    ones16 = jnp.ones((CHUNK, 16), jnp.float32)

    dp = _deg_kernel(dst_r, zeros16, ones16)
    da, db = dp[0], dp[1]

    y1 = _tc_call(_tc1_body, jax.ShapeDtypeStruct((NPAD, 64), jnp.float32),
                  x_p, W1, da, db)
    z1 = _scatter64(src_r, dst_r, y1, zeros64)
    y2 = _tc_call(_tc2_body, jax.ShapeDtypeStruct((NPAD, 48), jnp.float32),
                  z1[0], z1[1], y1, da, db, w2_p, b1_r)
    z2 = _scatter48(src_r, dst_r, y2, zeros48)
    out = _tc_call(_tc3_body, jax.ShapeDtypeStruct((N, 40), jnp.float32),
                   z2[0], z2[1], y2, da, db, b2_r)
    return out


# speedup vs baseline: 2.0921x; 1.1175x over previous
import functools

import jax
import jax.numpy as jnp
from jax import lax
from jax.experimental import pallas as pl
from jax.experimental.pallas import tpu as pltpu
from jax.experimental.pallas import tpu_sc as plsc

N = 10000
E = 320000
NPAD = 10016
DEGW = 8
NC, NS = 2, 16
NW = NC * NS
CHUNK = 128
NBUF = 2
LAG = 1
K1 = 80
EPT = K1 * CHUNK
EPAD = EPT * NW

_mesh = plsc.VectorSubcoreMesh(
    core_axis_name="c", subcore_axis_name="s", num_cores=NC, num_subcores=NS)
_sc_params = pltpu.CompilerParams(use_tc_tiling_on_sc=False)


def _deg_body(dst_hbm, zeros_hbm, ones_hbm, out_hbm, idx_v, ones_v, acc_sh, sem):
    cid = lax.axis_index("c")
    sid = lax.axis_index("s")
    wid = cid * NS + sid
    pltpu.sync_copy(dst_hbm.at[wid], idx_v)
    pltpu.sync_copy(ones_hbm, ones_v)

    @pl.when(sid == 0)
    def _zero():
        pltpu.sync_copy(zeros_hbm, acc_sh)

    plsc.subcore_barrier()

    def body(j, carry):
        pltpu.async_copy(ones_v, acc_sh.at[idx_v.at[j]], sem, add=True)
        return carry

    lax.fori_loop(0, K1, body, 0)

    def drain(j, carry):
        pltpu.make_async_copy(ones_v, acc_sh.at[idx_v.at[0]], sem).wait()
        return carry

    lax.fori_loop(0, K1, drain, 0)
    plsc.subcore_barrier()

    @pl.when(sid == 0)
    def _flush():
        pltpu.sync_copy(acc_sh, out_hbm.at[cid])


_deg_kernel = functools.partial(
    pl.kernel,
    out_type=jax.ShapeDtypeStruct((NC, NPAD, DEGW), jnp.float32),
    mesh=_mesh,
    compiler_params=_sc_params,
    scratch_types=[
        pltpu.VMEM((K1, CHUNK), jnp.int32),
        pltpu.VMEM((CHUNK, DEGW), jnp.float32),
        pltpu.VMEM_SHARED((NPAD, DEGW), jnp.float32),
        pltpu.SemaphoreType.DMA,
    ],
)(_deg_body)


def _make_scatter(D):
    def body(src_hbm, dst_hbm, y_hbm, zeros_hbm, out_hbm,
             src_v, dst_v, rows_v, acc_sh, y_sh, *sems):
        gsems, ssems = sems[:NBUF], sems[NBUF:]
        cid = lax.axis_index("c")
        sid = lax.axis_index("s")
        wid = cid * NS + sid
        pltpu.sync_copy(src_hbm.at[wid], src_v)
        pltpu.sync_copy(dst_hbm.at[wid], dst_v)

        @pl.when(jnp.logical_and(sid == 0, cid == 0))
        def _init0():
            pltpu.sync_copy(y_hbm, acc_sh)

        @pl.when(jnp.logical_and(sid == 0, cid == 1))
        def _init1():
            pltpu.sync_copy(zeros_hbm, acc_sh)

        @pl.when(sid == 1)
        def _stage():
            pltpu.sync_copy(y_hbm, y_sh)

        plsc.subcore_barrier()
        for g in range(LAG):
            pltpu.async_copy(y_sh.at[src_v.at[g]], rows_v.at[g], gsems[g])

        def step(g, carry):
            for b in range(NBUF):
                j = g * NBUF + b
                pltpu.make_async_copy(
                    y_sh.at[src_v.at[j]], rows_v.at[b], gsems[b]).wait()
                pltpu.async_copy(
                    rows_v.at[b], acc_sh.at[dst_v.at[j]], ssems[b], add=True)
                b2 = (b + LAG) % NBUF

                @pl.when(j + LAG < K1)
                def _refill():
                    @pl.when(j - LAG >= 0)
                    def _wait_prev_scatter():
                        pltpu.make_async_copy(
                            rows_v.at[b2], acc_sh.at[dst_v.at[0]],
                            ssems[b2]).wait()

                    pltpu.async_copy(
                        y_sh.at[src_v.at[j + LAG]], rows_v.at[b2], gsems[b2])
            return carry

        lax.fori_loop(0, K1 // NBUF, step, 0)
        for b in range(NBUF):
            pltpu.make_async_copy(
                rows_v.at[b], acc_sh.at[dst_v.at[0]], ssems[b]).wait()
        plsc.subcore_barrier()

        @pl.when(sid == 0)
        def _flush():
            pltpu.sync_copy(acc_sh, out_hbm.at[cid])

    return functools.partial(
        pl.kernel,
        out_type=jax.ShapeDtypeStruct((NC, NPAD, D), jnp.float32),
        mesh=_mesh,
        compiler_params=_sc_params,
        scratch_types=[
            pltpu.VMEM((K1, CHUNK), jnp.int32),
            pltpu.VMEM((K1, CHUNK), jnp.int32),
            pltpu.VMEM((NBUF, CHUNK, D), jnp.float32),
            pltpu.VMEM_SHARED((NPAD, D), jnp.float32),
            pltpu.VMEM_SHARED((NPAD, D), jnp.float32),
        ] + [pltpu.SemaphoreType.DMA] * (2 * NBUF),
    )(body)


_scatter64 = _make_scatter(64)
_scatter48 = _make_scatter(48)


def _dis(da_ref, db_ref):
    deg = da_ref[0, :, 0:1] + db_ref[0, :, 0:1] + 1.0
    return lax.rsqrt(deg)


_BLK = NPAD // 4
_BLK3 = N // 5


def _dp_specs():
    return [
        pl.BlockSpec((1, _BLK, DEGW), lambda g: (0, g, 0)),
        pl.BlockSpec((1, _BLK, DEGW), lambda g: (1, g, 0)),
    ]


def _tc1_body(x_ref, w_ref, da_ref, db_ref, y_ref):
    g = pl.program_id(0)
    dis = _dis(da_ref, db_ref)
    xw = jnp.dot(x_ref[...], w_ref[...], preferred_element_type=jnp.float32)
    rows = g * _BLK + lax.broadcasted_iota(jnp.int32, (_BLK, 1), 0)
    y_ref[...] = jnp.where(rows < N, xw * dis, 0.0)


_tc1 = pl.pallas_call(
    _tc1_body,
    grid=(4,),
    in_specs=[
        pl.BlockSpec((_BLK, 128), lambda g: (g, 0)),
        pl.BlockSpec((128, 64), lambda g: (0, 0)),
    ] + _dp_specs(),
    out_specs=pl.BlockSpec((_BLK, 64), lambda g: (g, 0)),
    out_shape=jax.ShapeDtypeStruct((NPAD, 64), jnp.float32),
)


def _tc2_body(z_ref, z_ref2, da_ref, db_ref, w_ref, b1_ref, y2_ref):
    g = pl.program_id(0)
    dis = _dis(da_ref, db_ref)
    pre = (z_ref[0] + z_ref2[0]) * dis + b1_ref[...]
    h = jnp.maximum(pre, 0.0)
    rows = g * _BLK + lax.broadcasted_iota(jnp.int32, (_BLK, 1), 0)
    h = jnp.where(rows < N, h, 0.0)
    y2_ref[...] = jnp.dot(h, w_ref[...], preferred_element_type=jnp.float32) * dis


_tc2 = pl.pallas_call(
    _tc2_body,
    grid=(4,),
    in_specs=[
        pl.BlockSpec((1, _BLK, 64), lambda g: (0, g, 0)),
        pl.BlockSpec((1, _BLK, 64), lambda g: (1, g, 0)),
    ] + _dp_specs() + [
        pl.BlockSpec((64, 48), lambda g: (0, 0)),
        pl.BlockSpec((1, 64), lambda g: (0, 0)),
    ],
    out_specs=pl.BlockSpec((_BLK, 48), lambda g: (g, 0)),
    out_shape=jax.ShapeDtypeStruct((NPAD, 48), jnp.float32),
)


def _tc3_body(z_ref, z_ref2, da_ref, db_ref, b2_ref, out_ref):
    dis = _dis(da_ref, db_ref)
    o = (z_ref[0] + z_ref2[0]) * dis + b2_ref[...]
    cols = lax.broadcasted_iota(jnp.int32, (_BLK3, 48), 1)
    valid = cols < 40
    m = jnp.max(jnp.where(valid, o, -jnp.inf), axis=1, keepdims=True)
    e = jnp.where(valid, jnp.exp(o - m), 0.0)
    s = jnp.sum(e, axis=1, keepdims=True)
    ls = o - m - jnp.log(s)
    out_ref[...] = ls[:, :40]


def _dp_specs3():
    return [
        pl.BlockSpec((1, _BLK3, DEGW), lambda g: (0, g, 0)),
        pl.BlockSpec((1, _BLK3, DEGW), lambda g: (1, g, 0)),
    ]


_tc3 = pl.pallas_call(
    _tc3_body,
    grid=(5,),
    in_specs=[
        pl.BlockSpec((1, _BLK3, 48), lambda g: (0, g, 0)),
        pl.BlockSpec((1, _BLK3, 48), lambda g: (1, g, 0)),
    ] + _dp_specs3() + [
        pl.BlockSpec((1, 48), lambda g: (0, 0)),
    ],
    out_specs=pl.BlockSpec((_BLK3, 40), lambda g: (g, 0)),
    out_shape=jax.ShapeDtypeStruct((N, 40), jnp.float32),
)


def kernel(x, edge_index, W1, b1, W2, b2):
    src = edge_index[0].astype(jnp.int32)
    dst = edge_index[1].astype(jnp.int32)
    pad = jnp.full((EPAD - E,), N, dtype=jnp.int32)
    src_r = jnp.concatenate([src, pad]).reshape(NW, K1, CHUNK)
    dst_r = jnp.concatenate([dst, pad]).reshape(NW, K1, CHUNK)

    w2_p = jnp.pad(W2, ((0, 0), (0, 8)))
    b1_r = b1.reshape(1, 64)
    b2_r = jnp.pad(b2, (0, 8)).reshape(1, 48)
    zeros8 = jnp.zeros((NPAD, DEGW), jnp.float32)
    zeros64 = jnp.zeros((NPAD, 64), jnp.float32)
    zeros48 = jnp.zeros((NPAD, 48), jnp.float32)
    ones8 = jnp.ones((CHUNK, DEGW), jnp.float32)

    dp = _deg_kernel(dst_r, zeros8, ones8)
    y1 = _tc1(x, W1, dp, dp)
    z1 = _scatter64(src_r, dst_r, y1, zeros64)
    y2 = _tc2(z1, z1, dp, dp, w2_p, b1_r)
    z2 = _scatter48(src_r, dst_r, y2, zeros48)
    out = _tc3(z2, z2, dp, dp, b2_r)
    return out


# speedup vs baseline: 2.1701x; 1.0373x over previous
import functools

import jax
import jax.numpy as jnp
from jax import lax
from jax.experimental import pallas as pl
from jax.experimental.pallas import tpu as pltpu
from jax.experimental.pallas import tpu_sc as plsc

N = 10000
E = 320000
NPAD = 10240
DEGW = 8
NC, NS = 2, 16
NW = NC * NS
CHUNK = 128
NBUF = 2
LAG = 1
K1 = 80
NCHUNK = E // CHUNK
TCHUNK = K1 * NW

_mesh = plsc.VectorSubcoreMesh(
    core_axis_name="c", subcore_axis_name="s", num_cores=NC, num_subcores=NS)
_sc_params = pltpu.CompilerParams(use_tc_tiling_on_sc=False)


def _deg_body(se_hbm, zeros_hbm, ones_hbm, out_hbm, idx_v, ones_v, acc_sh, sem):
    cid = lax.axis_index("c")
    sid = lax.axis_index("s")
    wid = cid * NS + sid
    pltpu.sync_copy(se_hbm.at[pl.ds(wid * K1, K1)], idx_v)
    pltpu.sync_copy(ones_hbm, ones_v)

    @pl.when(sid == 0)
    def _zero():
        pltpu.sync_copy(zeros_hbm, acc_sh)

    plsc.subcore_barrier()

    def body(j, carry):
        pltpu.async_copy(ones_v, acc_sh.at[idx_v.at[j, 1]], sem, add=True)
        return carry

    lax.fori_loop(0, K1, body, 0)

    def drain(j, carry):
        pltpu.make_async_copy(ones_v, acc_sh.at[idx_v.at[0, 1]], sem).wait()
        return carry

    lax.fori_loop(0, K1, drain, 0)
    plsc.subcore_barrier()

    @pl.when(sid == 0)
    def _flush():
        pltpu.sync_copy(acc_sh, out_hbm.at[cid])


_deg_kernel = functools.partial(
    pl.kernel,
    out_type=jax.ShapeDtypeStruct((NC, NPAD, DEGW), jnp.float32),
    mesh=_mesh,
    compiler_params=_sc_params,
    scratch_types=[
        pltpu.VMEM((K1, 2, CHUNK), jnp.int32),
        pltpu.VMEM((CHUNK, DEGW), jnp.float32),
        pltpu.VMEM_SHARED((NPAD, DEGW), jnp.float32),
        pltpu.SemaphoreType.DMA,
    ],
)(_deg_body)


def _make_scatter(D):
    def body(se_hbm, y_hbm, zeros_hbm, out_hbm,
             se_v, rows_v, acc_sh, y_sh, *sems):
        gsems, ssems = sems[:NBUF], sems[NBUF:]
        cid = lax.axis_index("c")
        sid = lax.axis_index("s")
        wid = cid * NS + sid
        pltpu.sync_copy(se_hbm.at[pl.ds(wid * K1, K1)], se_v)

        @pl.when(jnp.logical_and(sid == 0, cid == 0))
        def _init0():
            pltpu.sync_copy(y_hbm, acc_sh)

        @pl.when(jnp.logical_and(sid == 0, cid == 1))
        def _init1():
            pltpu.sync_copy(zeros_hbm, acc_sh)

        @pl.when(sid == 1)
        def _stage():
            pltpu.sync_copy(y_hbm, y_sh)

        plsc.subcore_barrier()
        for g in range(LAG):
            pltpu.async_copy(y_sh.at[se_v.at[g, 0]], rows_v.at[g], gsems[g])

        def step(g, carry):
            for b in range(NBUF):
                j = g * NBUF + b
                pltpu.make_async_copy(
                    y_sh.at[se_v.at[j, 0]], rows_v.at[b], gsems[b]).wait()
                pltpu.async_copy(
                    rows_v.at[b], acc_sh.at[se_v.at[j, 1]], ssems[b], add=True)
                b2 = (b + LAG) % NBUF

                @pl.when(j + LAG < K1)
                def _refill():
                    @pl.when(j - LAG >= 0)
                    def _wait_prev_scatter():
                        pltpu.make_async_copy(
                            rows_v.at[b2], acc_sh.at[se_v.at[0, 1]],
                            ssems[b2]).wait()

                    pltpu.async_copy(
                        y_sh.at[se_v.at[j + LAG, 0]], rows_v.at[b2], gsems[b2])
            return carry

        lax.fori_loop(0, K1 // NBUF, step, 0)
        for b in range(NBUF):
            pltpu.make_async_copy(
                rows_v.at[b], acc_sh.at[se_v.at[0, 1]], ssems[b]).wait()
        plsc.subcore_barrier()

        @pl.when(sid == 0)
        def _flush():
            pltpu.sync_copy(acc_sh, out_hbm.at[cid])

    return functools.partial(
        pl.kernel,
        out_type=jax.ShapeDtypeStruct((NC, NPAD, D), jnp.float32),
        mesh=_mesh,
        compiler_params=_sc_params,
        scratch_types=[
            pltpu.VMEM((K1, 2, CHUNK), jnp.int32),
            pltpu.VMEM((NBUF, CHUNK, D), jnp.float32),
            pltpu.VMEM_SHARED((NPAD, D), jnp.float32),
            pltpu.VMEM_SHARED((NPAD, D), jnp.float32),
        ] + [pltpu.SemaphoreType.DMA] * (2 * NBUF),
    )(body)


_scatter64 = _make_scatter(64)
_scatter48 = _make_scatter(48)


def _dis(da_ref, db_ref):
    deg = da_ref[0, :, 0:1] + db_ref[0, :, 0:1] + 1.0
    return lax.rsqrt(deg)


_BLK = NPAD // 4
_BLK3 = N // 5


def _dp_specs():
    return [
        pl.BlockSpec((1, _BLK, DEGW), lambda g: (0, g, 0)),
        pl.BlockSpec((1, _BLK, DEGW), lambda g: (1, g, 0)),
    ]


def _tc1_body(x_ref, w_ref, da_ref, db_ref, y_ref):
    g = pl.program_id(0)
    dis = _dis(da_ref, db_ref)
    xw = jnp.dot(x_ref[...], w_ref[...], preferred_element_type=jnp.float32)
    rows = g * _BLK + lax.broadcasted_iota(jnp.int32, (_BLK, 1), 0)
    y_ref[...] = jnp.where(rows < N, xw * dis, 0.0)


_tc1 = pl.pallas_call(
    _tc1_body,
    grid=(4,),
    in_specs=[
        pl.BlockSpec((_BLK, 128), lambda g: (g, 0)),
        pl.BlockSpec((128, 64), lambda g: (0, 0)),
    ] + _dp_specs(),
    out_specs=pl.BlockSpec((_BLK, 64), lambda g: (g, 0)),
    out_shape=jax.ShapeDtypeStruct((NPAD, 64), jnp.float32),
)


def _tc2_body(z_ref, z_ref2, da_ref, db_ref, w_ref, b1_ref, y2_ref):
    g = pl.program_id(0)
    dis = _dis(da_ref, db_ref)
    pre = (z_ref[0] + z_ref2[0]) * dis + b1_ref[...]
    h = jnp.maximum(pre, 0.0)
    rows = g * _BLK + lax.broadcasted_iota(jnp.int32, (_BLK, 1), 0)
    h = jnp.where(rows < N, h, 0.0)
    y2_ref[...] = jnp.dot(h, w_ref[...], preferred_element_type=jnp.float32) * dis


_tc2 = pl.pallas_call(
    _tc2_body,
    grid=(4,),
    in_specs=[
        pl.BlockSpec((1, _BLK, 64), lambda g: (0, g, 0)),
        pl.BlockSpec((1, _BLK, 64), lambda g: (1, g, 0)),
    ] + _dp_specs() + [
        pl.BlockSpec((64, 48), lambda g: (0, 0)),
        pl.BlockSpec((1, 64), lambda g: (0, 0)),
    ],
    out_specs=pl.BlockSpec((_BLK, 48), lambda g: (g, 0)),
    out_shape=jax.ShapeDtypeStruct((NPAD, 48), jnp.float32),
)


def _tc3_body(z_ref, z_ref2, da_ref, db_ref, b2_ref, out_ref):
    dis = _dis(da_ref, db_ref)
    o = (z_ref[0] + z_ref2[0]) * dis + b2_ref[...]
    cols = lax.broadcasted_iota(jnp.int32, (_BLK3, 48), 1)
    valid = cols < 40
    m = jnp.max(jnp.where(valid, o, -jnp.inf), axis=1, keepdims=True)
    e = jnp.where(valid, jnp.exp(o - m), 0.0)
    s = jnp.sum(e, axis=1, keepdims=True)
    ls = o - m - jnp.log(s)
    out_ref[...] = ls[:, :40]


def _dp_specs3():
    return [
        pl.BlockSpec((1, _BLK3, DEGW), lambda g: (0, g, 0)),
        pl.BlockSpec((1, _BLK3, DEGW), lambda g: (1, g, 0)),
    ]


_tc3 = pl.pallas_call(
    _tc3_body,
    grid=(5,),
    in_specs=[
        pl.BlockSpec((1, _BLK3, 48), lambda g: (0, g, 0)),
        pl.BlockSpec((1, _BLK3, 48), lambda g: (1, g, 0)),
    ] + _dp_specs3() + [
        pl.BlockSpec((1, 48), lambda g: (0, 0)),
    ],
    out_specs=pl.BlockSpec((_BLK3, 40), lambda g: (g, 0)),
    out_shape=jax.ShapeDtypeStruct((N, 40), jnp.float32),
)


def kernel(x, edge_index, W1, b1, W2, b2):
    se0 = edge_index.astype(jnp.int32).reshape(2, NCHUNK, CHUNK).transpose(1, 0, 2)
    se_pad = jnp.full((TCHUNK - NCHUNK, 2, CHUNK), N, dtype=jnp.int32)
    se = jnp.concatenate([se0, se_pad], axis=0)

    w2_p = jnp.pad(W2, ((0, 0), (0, 8)))
    b1_r = b1.reshape(1, 64)
    b2_r = jnp.pad(b2, (0, 8)).reshape(1, 48)
    zeros8 = jnp.zeros((NPAD, DEGW), jnp.float32)
    zeros64 = jnp.zeros((NPAD, 64), jnp.float32)
    zeros48 = jnp.zeros((NPAD, 48), jnp.float32)
    ones8 = jnp.ones((CHUNK, DEGW), jnp.float32)

    dp = _deg_kernel(se, zeros8, ones8)
    y1 = _tc1(x, W1, dp, dp)
    z1 = _scatter64(se, y1, zeros64)
    y2 = _tc2(z1, z1, dp, dp, w2_p, b1_r)
    z2 = _scatter48(se, y2, zeros48)
    out = _tc3(z2, z2, dp, dp, b2_r)
    return out
